# Initial kernel scaffold; baseline (speedup 1.0000x reference)
import functools

import jax
import jax.numpy as jnp
from jax import lax
from jax.experimental import pallas as pl
from jax.experimental.pallas import tpu as pltpu
from jax.experimental.pallas import tpu_sc as plsc

N_NODES = 50000
DIM = 64
HALF = 25000
ACC_ROWS = 25088
NSUB = 16
CH = 128
ZROWS = 128
ZCHUNK = ACC_ROWS // NSUB
CB_MAIN = HALF // NSUB // 8 * 8
CB_REM = HALF - CB_MAIN * NSUB

_mesh = functools.partial(
    plsc.VectorSubcoreMesh, core_axis_name="c", subcore_axis_name="s"
)


def _remap_dst(dst_v, loc_v, half_base):
    for j in range(CH // 16):
        v = dst_v[pl.ds(j * 16, 16)]
        local = v - half_base
        ok = (local >= 0) & (local < HALF)
        loc_v[pl.ds(j * 16, 16)] = jnp.where(ok, local, HALF)


def _zero_fill(buf, width):
    zero = jnp.zeros((16,), jnp.float32)
    if width == 0:
        for j in range(ZROWS // 16):
            buf[pl.ds(j * 16, 16)] = zero
    else:
        def body(i, _):
            for j in range(width // 16):
                buf[i, pl.ds(j * 16, 16)] = zero
            return 0
        lax.fori_loop(0, ZROWS, body, 0)


def _zero_spmem(zbuf, acc_sh, sid):
    z0 = sid * ZCHUNK
    for k in range(ZCHUNK // ZROWS):
        pltpu.sync_copy(zbuf, acc_sh.at[pl.ds(z0 + k * ZROWS, ZROWS)])
    rem = ZCHUNK % ZROWS
    if rem:
        pltpu.sync_copy(
            zbuf.at[pl.ds(0, rem)],
            acc_sh.at[pl.ds(z0 + ZCHUNK - rem, rem)],
        )


def _copy_back(acc_sh, out_hbm, cid, sid):
    r0 = sid * CB_MAIN
    base = cid * HALF
    pltpu.sync_copy(
        acc_sh.at[pl.ds(r0, CB_MAIN)], out_hbm.at[pl.ds(base + r0, CB_MAIN)]
    )
    if CB_REM:
        @pl.when(sid == 0)
        def _():
            pltpu.sync_copy(
                acc_sh.at[pl.ds(CB_MAIN * NSUB, CB_REM)],
                out_hbm.at[pl.ds(base + CB_MAIN * NSUB, CB_REM)],
            )


@functools.cache
def _make_deg_kernel(e_pad):
    nsteps = e_pad // NSUB // CH

    def body(dst_hbm, deg_hbm, dst_v, loc_v, ones_v, zbuf, deg_sh, sem):
        cid = lax.axis_index("c")
        sid = lax.axis_index("s")
        half_base = cid * HALF
        _zero_fill(zbuf, 0)
        one = jnp.full((16,), 1.0, jnp.float32)
        for j in range(CH // 16):
            ones_v[pl.ds(j * 16, 16)] = one
        _zero_spmem(zbuf, deg_sh, sid)
        plsc.subcore_barrier()
        ebase = sid * (e_pad // NSUB)

        def step(i, _):
            b = ebase + i * CH
            pltpu.sync_copy(dst_hbm.at[pl.ds(b, CH)], dst_v)
            _remap_dst(dst_v, loc_v, half_base)
            pltpu.sync_copy(ones_v, deg_sh.at[loc_v], add=True)
            return 0

        lax.fori_loop(0, nsteps, step, 0)
        plsc.subcore_barrier()
        _copy_back(deg_sh, deg_hbm, cid, sid)

    return pl.kernel(
        body,
        out_type=jax.ShapeDtypeStruct((N_NODES,), jnp.float32),
        mesh=_mesh(),
        scratch_types=[
            pltpu.VMEM((CH,), jnp.int32),
            pltpu.VMEM((CH,), jnp.int32),
            pltpu.VMEM((CH,), jnp.float32),
            pltpu.VMEM((ZROWS,), jnp.float32),
            pltpu.VMEM_SHARED((ACC_ROWS,), jnp.float32),
            pltpu.SemaphoreType.DMA,
        ],
    )


@functools.cache
def _make_agg_kernel(e_pad):
    nsteps = e_pad // NSUB // CH

    def body(src_hbm, dst_hbm, tab_hbm, agg_hbm,
             src_v, dst_v, loc_v, rows_v, zbuf, acc_sh, sem):
        cid = lax.axis_index("c")
        sid = lax.axis_index("s")
        half_base = cid * HALF
        _zero_fill(zbuf, DIM)
        _zero_spmem(zbuf, acc_sh, sid)
        plsc.subcore_barrier()
        ebase = sid * (e_pad // NSUB)

        def step(i, _):
            b = ebase + i * CH
            pltpu.sync_copy(src_hbm.at[pl.ds(b, CH)], src_v)
            pltpu.sync_copy(dst_hbm.at[pl.ds(b, CH)], dst_v)
            gather = pltpu.async_copy(tab_hbm.at[src_v], rows_v, sem)
            _remap_dst(dst_v, loc_v, half_base)
            gather.wait()
            pltpu.sync_copy(rows_v, acc_sh.at[loc_v], add=True)
            return 0

        lax.fori_loop(0, nsteps, step, 0)
        plsc.subcore_barrier()
        _copy_back(acc_sh, agg_hbm, cid, sid)

    return pl.kernel(
        body,
        out_type=jax.ShapeDtypeStruct((N_NODES, DIM), jnp.float32),
        mesh=_mesh(),
        scratch_types=[
            pltpu.VMEM((CH,), jnp.int32),
            pltpu.VMEM((CH,), jnp.int32),
            pltpu.VMEM((CH,), jnp.int32),
            pltpu.VMEM((CH, DIM), jnp.float32),
            pltpu.VMEM((ZROWS, DIM), jnp.float32),
            pltpu.VMEM_SHARED((ACC_ROWS, DIM), jnp.float32),
            pltpu.SemaphoreType.DMA,
        ],
    )


_BLK = 400
_GRID = N_NODES // _BLK

def _vec_spec():
    return pl.BlockSpec((_BLK, 1), lambda i: (i, 0))

def _mat_spec():
    return pl.BlockSpec((_BLK, DIM), lambda i: (i, 0))


def _scale_body(deg_ref, emb_ref, dis_ref, s0_ref):
    d = deg_ref[...]
    dis = jnp.where(d > 0, lax.rsqrt(d), 0.0)
    dis_ref[...] = dis
    s0_ref[...] = dis * emb_ref[...]


_scale_call = pl.pallas_call(
    _scale_body,
    grid=(_GRID,),
    in_specs=[_vec_spec(), _mat_spec()],
    out_specs=[_vec_spec(), _mat_spec()],
    out_shape=[
        jax.ShapeDtypeStruct((N_NODES, 1), jnp.float32),
        jax.ShapeDtypeStruct((N_NODES, DIM), jnp.float32),
    ],
)


def _mid_body(agg_ref, dis_ref, acc_ref, acc_out_ref, s_out_ref):
    agg = agg_ref[...]
    dis = dis_ref[...]
    scaled = dis * agg
    acc_out_ref[...] = acc_ref[...] + scaled
    s_out_ref[...] = dis * scaled


_mid_call = pl.pallas_call(
    _mid_body,
    grid=(_GRID,),
    in_specs=[_mat_spec(), _vec_spec(), _mat_spec()],
    out_specs=[_mat_spec(), _mat_spec()],
    out_shape=[
        jax.ShapeDtypeStruct((N_NODES, DIM), jnp.float32),
        jax.ShapeDtypeStruct((N_NODES, DIM), jnp.float32),
    ],
)


def _last_body(agg_ref, dis_ref, acc_ref, out_ref):
    out_ref[...] = (acc_ref[...] + dis_ref[...] * agg_ref[...]) * 0.25


_last_call = pl.pallas_call(
    _last_body,
    grid=(_GRID,),
    in_specs=[_mat_spec(), _vec_spec(), _mat_spec()],
    out_specs=_mat_spec(),
    out_shape=jax.ShapeDtypeStruct((N_NODES, DIM), jnp.float32),
)


def kernel(edge_index, emb_weight):
    src = edge_index[0]
    dst = edge_index[1]
    n_edges = src.shape[0]
    e_pad = -(-n_edges // (NSUB * CH)) * (NSUB * CH)
    pad = e_pad - n_edges
    if pad:
        src = jnp.concatenate([src, jnp.zeros((pad,), jnp.int32)])
        dst = jnp.concatenate([dst, jnp.full((pad,), -1, jnp.int32)])

    deg = _make_deg_kernel(e_pad)(dst).reshape(N_NODES, 1)
    dis, s = _scale_call(deg, emb_weight)
    acc = emb_weight
    agg_kernel = _make_agg_kernel(e_pad)
    for layer in range(3):
        agg = agg_kernel(src, dst, s)
        if layer < 2:
            acc, s = _mid_call(agg, dis, acc)
        else:
            out = _last_call(agg, dis, acc)
    return (emb_weight, out)


# speedup vs baseline: 6.4193x; 6.4193x over previous
import functools

import jax
import jax.numpy as jnp
from jax import lax
from jax.experimental import pallas as pl
from jax.experimental.pallas import tpu as pltpu
from jax.experimental.pallas import tpu_sc as plsc

N_NODES = 50000
DIM = 64
HALF = 25000
ACC_ROWS = 25088
NSUB = 16
CH = 128
ZROWS = 128
ZCHUNK = ACC_ROWS // NSUB
CB_MAIN = HALF // NSUB // 8 * 8
CB_REM = HALF - CB_MAIN * NSUB

_mesh = functools.partial(
    plsc.VectorSubcoreMesh, core_axis_name="c", subcore_axis_name="s"
)


def _remap_dst(dst_v, loc_v, half_base):
    for j in range(CH // 16):
        v = dst_v[pl.ds(j * 16, 16)]
        local = v - half_base
        ok = (local >= 0) & (local < HALF)
        loc_v[pl.ds(j * 16, 16)] = jnp.where(ok, local, HALF)


def _zero_fill(buf, width):
    zero = jnp.zeros((16,), jnp.float32)
    if width == 0:
        for j in range(ZROWS // 16):
            buf[pl.ds(j * 16, 16)] = zero
    else:
        def body(i, _):
            for j in range(width // 16):
                buf[i, pl.ds(j * 16, 16)] = zero
            return 0
        lax.fori_loop(0, ZROWS, body, 0)


def _zero_spmem(zbuf, acc_sh, sid):
    z0 = sid * ZCHUNK
    for k in range(ZCHUNK // ZROWS):
        pltpu.sync_copy(zbuf, acc_sh.at[pl.ds(z0 + k * ZROWS, ZROWS)])
    rem = ZCHUNK % ZROWS
    if rem:
        pltpu.sync_copy(
            zbuf.at[pl.ds(0, rem)],
            acc_sh.at[pl.ds(z0 + ZCHUNK - rem, rem)],
        )


def _copy_back(acc_sh, out_hbm, cid, sid, sbuf):
    base = cid * HALF

    def move(row0, n):
        pltpu.sync_copy(acc_sh.at[pl.ds(row0, n)], sbuf.at[pl.ds(0, n)])
        pltpu.sync_copy(sbuf.at[pl.ds(0, n)], out_hbm.at[pl.ds(base + row0, n)])

    r0 = sid * CB_MAIN
    for k in range(CB_MAIN // ZROWS):
        move(r0 + k * ZROWS, ZROWS)
    rem = CB_MAIN % ZROWS
    if rem:
        move(r0 + CB_MAIN - rem, rem)
    if CB_REM:
        @pl.when(sid == 0)
        def _():
            move(CB_MAIN * NSUB, CB_REM)


@functools.cache
def _make_deg_kernel(e_pad):
    nsteps = e_pad // NSUB // CH

    def body(dst_hbm, deg_hbm, dst_v, loc_v, ones_v, zbuf, deg_sh, sem):
        cid = lax.axis_index("c")
        sid = lax.axis_index("s")
        half_base = cid * HALF
        _zero_fill(zbuf, 0)
        one = jnp.full((16,), 1.0, jnp.float32)
        for j in range(CH // 16):
            ones_v[pl.ds(j * 16, 16)] = one
        _zero_spmem(zbuf, deg_sh, sid)
        plsc.subcore_barrier()
        ebase = sid * (e_pad // NSUB)

        def step(i, _):
            b = ebase + i * CH
            pltpu.sync_copy(dst_hbm.at[pl.ds(b, CH)], dst_v)
            _remap_dst(dst_v, loc_v, half_base)
            pltpu.sync_copy(ones_v, deg_sh.at[loc_v], add=True)
            return 0

        lax.fori_loop(0, nsteps, step, 0)
        plsc.subcore_barrier()
        _copy_back(deg_sh, deg_hbm, cid, sid, zbuf)

    return pl.kernel(
        body,
        out_type=jax.ShapeDtypeStruct((N_NODES,), jnp.float32),
        mesh=_mesh(),
        compiler_params=pltpu.CompilerParams(use_tc_tiling_on_sc=False),
        scratch_types=[
            pltpu.VMEM((CH,), jnp.int32),
            pltpu.VMEM((CH,), jnp.int32),
            pltpu.VMEM((CH,), jnp.float32),
            pltpu.VMEM((ZROWS,), jnp.float32),
            pltpu.VMEM_SHARED((ACC_ROWS,), jnp.float32),
            pltpu.SemaphoreType.DMA,
        ],
    )


@functools.cache
def _make_agg_kernel(e_pad):
    nsteps = e_pad // NSUB // CH

    def body(src_hbm, dst_hbm, tab_hbm, agg_hbm,
             src_v, dst_v, loc_v, rows_v, zbuf, acc_sh, sem):
        cid = lax.axis_index("c")
        sid = lax.axis_index("s")
        half_base = cid * HALF
        _zero_fill(zbuf, DIM)
        _zero_spmem(zbuf, acc_sh, sid)
        plsc.subcore_barrier()
        ebase = sid * (e_pad // NSUB)

        def step(i, _):
            b = ebase + i * CH
            pltpu.sync_copy(src_hbm.at[pl.ds(b, CH)], src_v)
            pltpu.sync_copy(dst_hbm.at[pl.ds(b, CH)], dst_v)
            gather = pltpu.async_copy(tab_hbm.at[src_v], rows_v, sem)
            _remap_dst(dst_v, loc_v, half_base)
            gather.wait()
            pltpu.sync_copy(rows_v, acc_sh.at[loc_v], add=True)
            return 0

        lax.fori_loop(0, nsteps, step, 0)
        plsc.subcore_barrier()
        _copy_back(acc_sh, agg_hbm, cid, sid, zbuf)

    return pl.kernel(
        body,
        out_type=jax.ShapeDtypeStruct((N_NODES, DIM), jnp.float32),
        mesh=_mesh(),
        compiler_params=pltpu.CompilerParams(use_tc_tiling_on_sc=False),
        scratch_types=[
            pltpu.VMEM((CH,), jnp.int32),
            pltpu.VMEM((CH,), jnp.int32),
            pltpu.VMEM((CH,), jnp.int32),
            pltpu.VMEM((CH, DIM), jnp.float32),
            pltpu.VMEM((ZROWS, DIM), jnp.float32),
            pltpu.VMEM_SHARED((ACC_ROWS, DIM), jnp.float32),
            pltpu.SemaphoreType.DMA,
        ],
    )


_BLK = 400
_GRID = N_NODES // _BLK

def _vec_spec():
    return pl.BlockSpec((_BLK, 1), lambda i: (i, 0))

def _mat_spec():
    return pl.BlockSpec((_BLK, DIM), lambda i: (i, 0))


def _scale_body(deg_ref, emb_ref, dis_ref, s0_ref):
    d = deg_ref[...]
    dis = jnp.where(d > 0, lax.rsqrt(d), 0.0)
    dis_ref[...] = dis
    s0_ref[...] = dis * emb_ref[...]


_scale_call = pl.pallas_call(
    _scale_body,
    grid=(_GRID,),
    in_specs=[_vec_spec(), _mat_spec()],
    out_specs=[_vec_spec(), _mat_spec()],
    out_shape=[
        jax.ShapeDtypeStruct((N_NODES, 1), jnp.float32),
        jax.ShapeDtypeStruct((N_NODES, DIM), jnp.float32),
    ],
)


def _mid_body(agg_ref, dis_ref, acc_ref, acc_out_ref, s_out_ref):
    agg = agg_ref[...]
    dis = dis_ref[...]
    scaled = dis * agg
    acc_out_ref[...] = acc_ref[...] + scaled
    s_out_ref[...] = dis * scaled


_mid_call = pl.pallas_call(
    _mid_body,
    grid=(_GRID,),
    in_specs=[_mat_spec(), _vec_spec(), _mat_spec()],
    out_specs=[_mat_spec(), _mat_spec()],
    out_shape=[
        jax.ShapeDtypeStruct((N_NODES, DIM), jnp.float32),
        jax.ShapeDtypeStruct((N_NODES, DIM), jnp.float32),
    ],
)


def _last_body(agg_ref, dis_ref, acc_ref, out_ref):
    out_ref[...] = (acc_ref[...] + dis_ref[...] * agg_ref[...]) * 0.25


_last_call = pl.pallas_call(
    _last_body,
    grid=(_GRID,),
    in_specs=[_mat_spec(), _vec_spec(), _mat_spec()],
    out_specs=_mat_spec(),
    out_shape=jax.ShapeDtypeStruct((N_NODES, DIM), jnp.float32),
)


def kernel(edge_index, emb_weight):
    src = edge_index[0]
    dst = edge_index[1]
    n_edges = src.shape[0]
    e_pad = -(-n_edges // (NSUB * CH)) * (NSUB * CH)
    pad = e_pad - n_edges
    if pad:
        src = jnp.concatenate([src, jnp.zeros((pad,), jnp.int32)])
        dst = jnp.concatenate([dst, jnp.full((pad,), -1, jnp.int32)])

    deg = _make_deg_kernel(e_pad)(dst).reshape(N_NODES, 1)
    dis, s = _scale_call(deg, emb_weight)
    acc = emb_weight
    agg_kernel = _make_agg_kernel(e_pad)
    for layer in range(3):
        agg = agg_kernel(src, dst, s)
        if layer < 2:
            acc, s = _mid_call(agg, dis, acc)
        else:
            out = _last_call(agg, dis, acc)
    return (emb_weight, out)


# speedup vs baseline: 8.5064x; 1.3251x over previous
import functools

import jax
import jax.numpy as jnp
from jax import lax
from jax.experimental import pallas as pl
from jax.experimental.pallas import tpu as pltpu
from jax.experimental.pallas import tpu_sc as plsc

N_NODES = 50000
DIM = 64
HALF = 25000
ACC_ROWS = 25008
DEG_ROWS = 25088
NSUB = 16
CH = 128
ZROWS = 128
CB_MAIN = HALF // NSUB // 8 * 8
CB_REM = HALF - CB_MAIN * NSUB

_mesh = functools.partial(
    plsc.VectorSubcoreMesh, core_axis_name="c", subcore_axis_name="s"
)


def _remap_dst(dst_v, loc_v, half_base):
    for j in range(CH // 16):
        v = dst_v[pl.ds(j * 16, 16)]
        local = v - half_base
        ok = (local >= 0) & (local < HALF)
        loc_v[pl.ds(j * 16, 16)] = jnp.where(ok, local, HALF)


def _zero_fill(buf, width):
    zero = jnp.zeros((16,), jnp.float32)
    if width == 0:
        for j in range(ZROWS // 16):
            buf[pl.ds(j * 16, 16)] = zero
    else:
        def body(i, _):
            for j in range(width // 16):
                buf[i, pl.ds(j * 16, 16)] = zero
            return 0
        lax.fori_loop(0, ZROWS, body, 0)


def _zero_spmem(zbuf, acc_sh, sid, nrows):
    chunk = nrows // NSUB
    z0 = sid * chunk
    for k in range(chunk // ZROWS):
        pltpu.sync_copy(zbuf, acc_sh.at[pl.ds(z0 + k * ZROWS, ZROWS)])
    rem = chunk % ZROWS
    if rem:
        pltpu.sync_copy(
            zbuf.at[pl.ds(0, rem)],
            acc_sh.at[pl.ds(z0 + chunk - rem, rem)],
        )


def _chunked_back(move, sid):
    r0 = sid * CB_MAIN

    def k_body(k, _):
        move(r0 + k * ZROWS, ZROWS)
        return 0

    lax.fori_loop(0, CB_MAIN // ZROWS, k_body, 0)
    rem = CB_MAIN % ZROWS
    if rem:
        move(r0 + CB_MAIN - rem, rem)
    if CB_REM:
        @pl.when(sid == 0)
        def _():
            move(CB_MAIN * NSUB, CB_REM)


def _copy_back(acc_sh, out_hbm, cid, sid, sbuf):
    base = cid * HALF

    def move(row0, n):
        pltpu.sync_copy(acc_sh.at[pl.ds(row0, n)], sbuf.at[pl.ds(0, n)])
        pltpu.sync_copy(sbuf.at[pl.ds(0, n)], out_hbm.at[pl.ds(base + row0, n)])

    _chunked_back(move, sid)


@functools.cache
def _make_deg_kernel(e_pad):
    ngroups = e_pad // CH // NSUB // K_SLOTS
    npairs = ngroups // 2
    assert ngroups % 2 == 0

    def body(dst_hbm, deg_hbm, *refs):
        dst_v = [[refs[p * K_SLOTS + b] for b in range(K_SLOTS)] for p in range(2)]
        o = 2 * K_SLOTS
        loc_v = [[refs[o + p * K_SLOTS + b] for b in range(K_SLOTS)] for p in range(2)]
        o += 2 * K_SLOTS
        ones_v, zbuf, deg_sh, isem, ssem = refs[o:]

        cid = lax.axis_index("c")
        sid = lax.axis_index("s")
        half_base = cid * HALF
        _zero_fill(zbuf, 0)
        one = jnp.full((16,), 1.0, jnp.float32)
        for j in range(CH // 16):
            ones_v[pl.ds(j * 16, 16)] = one
        _zero_spmem(zbuf, deg_sh, sid, DEG_ROWS)
        plsc.subcore_barrier()
        ebase = sid * (e_pad // NSUB)

        def idx_ref(g, b):
            return dst_hbm.at[pl.ds(ebase + (g * K_SLOTS + b) * CH, CH)]

        def idx_load(g, p, b):
            pltpu.async_copy(idx_ref(g, b), dst_v[p][b], isem)

        for b in range(K_SLOTS):
            idx_load(0, 0, b)

        def sc_desc(p, b):
            return pltpu.make_async_copy(ones_v, deg_sh.at[loc_v[p][b]], ssem)

        def half_body(g, p):
            for b in range(K_SLOTS):
                pltpu.make_async_copy(idx_ref(g, b), dst_v[p][b], isem).wait()
            @pl.when(g + 1 < ngroups)
            def _():
                for b in range(K_SLOTS):
                    idx_load(g + 1, 1 - p, b)
            @pl.when(g >= 2)
            def _():
                for b in range(K_SLOTS):
                    sc_desc(p, b).wait()
            for b in range(K_SLOTS):
                _remap_dst(dst_v[p][b], loc_v[p][b], half_base)
            for b in range(K_SLOTS):
                pltpu.async_copy(ones_v, deg_sh.at[loc_v[p][b]], ssem, add=True)

        def pair(i, _):
            half_body(2 * i, 0)
            half_body(2 * i + 1, 1)
            return 0

        lax.fori_loop(0, npairs, pair, 0)
        for p in range(2):
            for b in range(K_SLOTS):
                sc_desc(p, b).wait()
        plsc.subcore_barrier()
        _copy_back(deg_sh, deg_hbm, cid, sid, zbuf)

    scratch = (
        [pltpu.VMEM((CH,), jnp.int32) for _ in range(4 * K_SLOTS)]
        + [
            pltpu.VMEM((CH,), jnp.float32),
            pltpu.VMEM((ZROWS,), jnp.float32),
            pltpu.VMEM_SHARED((DEG_ROWS,), jnp.float32),
            pltpu.SemaphoreType.DMA,
            pltpu.SemaphoreType.DMA,
        ]
    )
    return pl.kernel(
        body,
        out_type=jax.ShapeDtypeStruct((N_NODES,), jnp.float32),
        mesh=_mesh(),
        compiler_params=pltpu.CompilerParams(use_tc_tiling_on_sc=False),
        scratch_types=scratch,
    )


K_SLOTS = 2


@functools.cache
def _make_agg_kernel(e_pad):
    nchunks = e_pad // CH
    npersub = nchunks // NSUB
    ngroups = npersub // K_SLOTS
    npairs = ngroups // 2
    assert npersub % K_SLOTS == 0 and ngroups % 2 == 0

    def body(ed_hbm, tab_hbm, agg_hbm, *refs):
        ed_v = [[refs[p * K_SLOTS + b] for b in range(K_SLOTS)] for p in range(2)]
        o = 2 * K_SLOTS
        loc_v = [[refs[o + p * K_SLOTS + b] for b in range(K_SLOTS)] for p in range(2)]
        o += 2 * K_SLOTS
        rows_v = [refs[o + b] for b in range(K_SLOTS)]
        o += K_SLOTS
        zbuf, acc_sh, isem, gsem0, gsem1, ssem = refs[o:]
        gsems = [gsem0, gsem1]

        cid = lax.axis_index("c")
        sid = lax.axis_index("s")
        half_base = cid * HALF
        _zero_fill(zbuf, DIM)
        _zero_spmem(zbuf, acc_sh, sid, ACC_ROWS)
        plsc.subcore_barrier()
        cbase = sid * npersub

        def idx_load(g, p, b):
            pltpu.async_copy(ed_hbm.at[cbase + g * K_SLOTS + b], ed_v[p][b], isem)

        for b in range(K_SLOTS):
            idx_load(0, 0, b)

        def sc_desc(p, b):
            return pltpu.make_async_copy(
                rows_v[b], acc_sh.at[loc_v[p][b]], ssem
            )

        def half_body(g, p):
            for b in range(K_SLOTS):
                pltpu.make_async_copy(
                    ed_hbm.at[cbase + g * K_SLOTS + b], ed_v[p][b], isem
                ).wait()
            @pl.when(g >= 1)
            def _():
                for b in range(K_SLOTS):
                    sc_desc(1 - p, b).wait()
            gathers = [
                pltpu.async_copy(
                    tab_hbm.at[ed_v[p][b].at[0]], rows_v[b], gsems[b]
                )
                for b in range(K_SLOTS)
            ]
            @pl.when(g + 1 < ngroups)
            def _():
                for b in range(K_SLOTS):
                    idx_load(g + 1, 1 - p, b)
            for b in range(K_SLOTS):
                for j in range(CH // 16):
                    v = ed_v[p][b][1, pl.ds(j * 16, 16)]
                    local = v - half_base
                    ok = (local >= 0) & (local < HALF)
                    loc_v[p][b][pl.ds(j * 16, 16)] = jnp.where(ok, local, HALF)
            for b in range(K_SLOTS):
                gathers[b].wait()
                pltpu.async_copy(
                    rows_v[b], acc_sh.at[loc_v[p][b]], ssem, add=True
                )

        def pair(i, _):
            half_body(2 * i, 0)
            half_body(2 * i + 1, 1)
            return 0

        lax.fori_loop(0, npairs, pair, 0)
        for b in range(K_SLOTS):
            sc_desc(1, b).wait()
        plsc.subcore_barrier()

        _copy_back(acc_sh, agg_hbm, cid, sid, zbuf)

    out_mat = jax.ShapeDtypeStruct((N_NODES, DIM), jnp.float32)
    scratch = (
        [pltpu.VMEM((2, CH), jnp.int32) for _ in range(2 * K_SLOTS)]
        + [pltpu.VMEM((CH,), jnp.int32) for _ in range(2 * K_SLOTS)]
        + [pltpu.VMEM((CH, DIM), jnp.float32) for _ in range(K_SLOTS)]
        + [
            pltpu.VMEM((ZROWS, DIM), jnp.float32),
            pltpu.VMEM_SHARED((ACC_ROWS, DIM), jnp.float32),
            pltpu.SemaphoreType.DMA,
            pltpu.SemaphoreType.DMA,
            pltpu.SemaphoreType.DMA,
            pltpu.SemaphoreType.DMA,
        ]
    )
    return pl.kernel(
        body,
        out_type=out_mat,
        mesh=_mesh(),
        compiler_params=pltpu.CompilerParams(use_tc_tiling_on_sc=False),
        scratch_types=scratch,
    )


_BLK = 400
_GRID = N_NODES // _BLK

def _vec_spec():
    return pl.BlockSpec((_BLK, 1), lambda i: (i, 0))

def _mat_spec():
    return pl.BlockSpec((_BLK, DIM), lambda i: (i, 0))


def _scale_body(deg_ref, emb_ref, dis_ref, s0_ref):
    d = deg_ref[...]
    dis = jnp.where(d > 0, lax.rsqrt(d), 0.0)
    dis_ref[...] = dis
    s0_ref[...] = dis * emb_ref[...]


_scale_call = pl.pallas_call(
    _scale_body,
    grid=(_GRID,),
    in_specs=[_vec_spec(), _mat_spec()],
    out_specs=[_vec_spec(), _mat_spec()],
    out_shape=[
        jax.ShapeDtypeStruct((N_NODES, 1), jnp.float32),
        jax.ShapeDtypeStruct((N_NODES, DIM), jnp.float32),
    ],
)


def _s_body(agg_ref, dis_ref, s_ref):
    dis = dis_ref[...]
    s_ref[...] = dis * dis * agg_ref[...]


_s_call = pl.pallas_call(
    _s_body,
    grid=(_GRID,),
    in_specs=[_mat_spec(), _vec_spec()],
    out_specs=_mat_spec(),
    out_shape=jax.ShapeDtypeStruct((N_NODES, DIM), jnp.float32),
)


def _final_body(emb_ref, dis_ref, a1_ref, a2_ref, a3_ref, out_ref):
    out_ref[...] = 0.25 * (
        emb_ref[...]
        + dis_ref[...] * (a1_ref[...] + a2_ref[...] + a3_ref[...])
    )


_final_call = pl.pallas_call(
    _final_body,
    grid=(_GRID,),
    in_specs=[_mat_spec(), _vec_spec()] + [_mat_spec()] * 3,
    out_specs=_mat_spec(),
    out_shape=jax.ShapeDtypeStruct((N_NODES, DIM), jnp.float32),
)


def kernel(edge_index, emb_weight):
    src = edge_index[0]
    dst = edge_index[1]
    n_edges = src.shape[0]
    grain = NSUB * CH * K_SLOTS * 2
    e_pad = -(-n_edges // grain) * grain
    pad = e_pad - n_edges
    if pad:
        src = jnp.concatenate([src, jnp.zeros((pad,), jnp.int32)])
        dst = jnp.concatenate([dst, jnp.full((pad,), -1, jnp.int32)])
    ed = jnp.stack([src, dst]).reshape(2, e_pad // CH, CH).transpose(1, 0, 2)

    deg = _make_deg_kernel(e_pad)(dst).reshape(N_NODES, 1)
    dis, s = _scale_call(deg, emb_weight)
    agg_kernel = _make_agg_kernel(e_pad)
    agg1 = agg_kernel(ed, s)
    s = _s_call(agg1, dis)
    agg2 = agg_kernel(ed, s)
    s = _s_call(agg2, dis)
    agg3 = agg_kernel(ed, s)
    out = _final_call(emb_weight, dis, agg1, agg2, agg3)
    return (emb_weight, out)
